# Initial kernel scaffold; baseline (speedup 1.0000x reference)
import jax
import jax.numpy as jnp
from jax.experimental import pallas as pl


def kernel(x_spt, edge_index_spt, batch_spt, y_spt, x_qry, edge_index_qry, batch_qry, y_qry, W1, b1, W2, b2, Wg, bg):
    raise NotImplementedError("write your pallas kernel here")


# speedup vs baseline: 4.3989x; 4.3989x over previous
import functools

import jax
import jax.numpy as jnp
from jax import lax
from jax.experimental import pallas as pl
from jax.experimental.pallas import tpu as pltpu
from jax.experimental.pallas import tpu_sc as plsc

EMB = 128
NLAYER = 5
LR = 0.01
NSTEP = 3
BLK = 1000

_HI = lax.Precision.HIGHEST


def _dot(a, b, ca, cb):
    return lax.dot_general(a, b, (((ca,), (cb,)), ((), ())),
                           precision=_HI, preferred_element_type=jnp.float32)


@functools.lru_cache(maxsize=None)
def _make_sc_spmm(N, E):
    NC, NS = 2, 16
    NW = NC * NS
    EW = E // NW
    K = 80
    NCHUNK = EW // K
    RPS = (N // NS) & ~7
    RLAST = N - (NS - 1) * RPS
    ZR = 16
    mesh = plsc.VectorSubcoreMesh(core_axis_name="c", subcore_axis_name="s")

    @functools.partial(
        pl.kernel,
        mesh=mesh,
        out_type=jax.ShapeDtypeStruct((NC, N, EMB), jnp.float32),
        scratch_types=[
            pltpu.VMEM((K,), jnp.int32),
            pltpu.VMEM((K,), jnp.int32),
            pltpu.VMEM((K, EMB), jnp.float32),
            pltpu.VMEM((ZR, EMB), jnp.float32),
            pltpu.VMEM_SHARED((N, EMB), jnp.float32),
            pltpu.SemaphoreType.DMA,
        ],
    )
    def spmm(h_hbm, src_hbm, dst_hbm, out_hbm, src_v, dst_v, rows_v, zbuf, acc, sem):
        c = lax.axis_index("c")
        s = lax.axis_index("s")
        wid = s * NC + c
        zero16 = jnp.zeros((16,), jnp.float32)
        for i in range(ZR):
            for j in range(EMB // 16):
                zbuf[i, pl.ds(j * 16, 16)] = zero16
        base = s * RPS

        @pl.when(s < NS - 1)
        def _zero_main():
            for i in range(RPS // ZR):
                pltpu.sync_copy(zbuf, acc.at[pl.ds(base + i * ZR, ZR)])

        @pl.when(s == NS - 1)
        def _zero_last():
            for i in range(RLAST // ZR):
                pltpu.sync_copy(zbuf, acc.at[pl.ds(base + i * ZR, ZR)])

        plsc.subcore_barrier()

        def body(i, carry):
            e0 = wid * EW + i * K
            pltpu.sync_copy(src_hbm.at[pl.ds(e0, K)], src_v)
            pltpu.sync_copy(dst_hbm.at[pl.ds(e0, K)], dst_v)
            pltpu.async_copy(h_hbm.at[src_v], rows_v, sem).wait()
            pltpu.sync_copy(rows_v, acc.at[dst_v], add=True)
            return carry

        lax.fori_loop(0, NCHUNK, body, 0)
        plsc.subcore_barrier()

        @pl.when(s < NS - 1)
        def _out_main():
            pltpu.sync_copy(acc.at[pl.ds(base, RPS)],
                            out_hbm.at[c, pl.ds(base, RPS)])

        @pl.when(s == NS - 1)
        def _out_last():
            pltpu.sync_copy(acc.at[pl.ds(base, RLAST)],
                            out_hbm.at[c, pl.ds(base, RLAST)])

    return spmm


def _sc_spmm(h, src, dst):
    N = h.shape[0]
    E = src.shape[0]
    return _make_sc_spmm(N, E)(h, src, dst)


@functools.lru_cache(maxsize=None)
def _make_fwd(N, last):
    NB = N // BLK

    def body(p0_ref, p1_ref, h_ref, w1_ref, b1_ref, w2_ref, b2_ref,
             agg_ref, r_ref, hn_ref):
        agg = p0_ref[0] + p1_ref[0] + h_ref[...]
        agg_ref[...] = agg
        z1 = _dot(agg, w1_ref[...], 1, 0) + b1_ref[...]
        r = jnp.maximum(z1, 0.0)
        r_ref[...] = r
        z2 = _dot(r, w2_ref[...], 1, 0) + b2_ref[...]
        hn_ref[...] = z2 if last else jnp.maximum(z2, 0.0)

    blk = pl.BlockSpec((BLK, EMB), lambda i: (i, 0))
    return pl.pallas_call(
        body,
        grid=(NB,),
        in_specs=[
            pl.BlockSpec((1, BLK, EMB), lambda i: (0, i, 0)),
            pl.BlockSpec((1, BLK, EMB), lambda i: (1, i, 0)),
            blk,
            pl.BlockSpec((EMB, EMB), lambda i: (0, 0)),
            pl.BlockSpec((1, EMB), lambda i: (0, 0)),
            pl.BlockSpec((EMB, EMB), lambda i: (0, 0)),
            pl.BlockSpec((1, EMB), lambda i: (0, 0)),
        ],
        out_specs=[blk, blk, blk],
        out_shape=[jax.ShapeDtypeStruct((N, EMB), jnp.float32)] * 3,
    )


@functools.lru_cache(maxsize=None)
def _make_bwd(N, last, combine, need_dagg):
    NB = N // BLK

    def body(*refs):
        refs = list(refs)
        if combine:
            q0_ref, q1_ref, dp_ref = refs[:3]
            refs = refs[3:]
            dh = q0_ref[0] + q1_ref[0] + dp_ref[...]
        else:
            dh = refs.pop(0)[...]
        if not last:
            hn_ref = refs.pop(0)
            dh = dh * (hn_ref[...] > 0).astype(jnp.float32)
        (r_ref, agg_ref, w1_ref, b1_ref, w2_ref, b2_ref) = refs[:6]
        outs = refs[6:]
        if need_dagg:
            dagg_ref = outs.pop(0)
        w1n_ref, b1n_ref, w2n_ref, b2n_ref, aW1, ab1, aW2, ab2 = outs
        i = pl.program_id(0)

        @pl.when(i == 0)
        def _init():
            aW1[...] = jnp.zeros((EMB, EMB), jnp.float32)
            ab1[...] = jnp.zeros((1, EMB), jnp.float32)
            aW2[...] = jnp.zeros((EMB, EMB), jnp.float32)
            ab2[...] = jnp.zeros((1, EMB), jnp.float32)

        r = r_ref[...]
        aW2[...] += _dot(r, dh, 0, 0)
        ab2[...] += jnp.sum(dh, axis=0, keepdims=True)
        dr = _dot(dh, w2_ref[...], 1, 1)
        dz1 = dr * (r > 0).astype(jnp.float32)
        aW1[...] += _dot(agg_ref[...], dz1, 0, 0)
        ab1[...] += jnp.sum(dz1, axis=0, keepdims=True)
        if need_dagg:
            dagg_ref[...] = _dot(dz1, w1_ref[...], 1, 1)

        @pl.when(i == NB - 1)
        def _finish():
            w1n_ref[...] = w1_ref[...] - LR * aW1[...]
            b1n_ref[...] = b1_ref[...] - LR * ab1[...]
            w2n_ref[...] = w2_ref[...] - LR * aW2[...]
            b2n_ref[...] = b2_ref[...] - LR * ab2[...]

    blk = pl.BlockSpec((BLK, EMB), lambda i: (i, 0))
    wspec = pl.BlockSpec((EMB, EMB), lambda i: (0, 0))
    bspec = pl.BlockSpec((1, EMB), lambda i: (0, 0))
    in_specs = []
    if combine:
        in_specs += [pl.BlockSpec((1, BLK, EMB), lambda i: (0, i, 0)),
                     pl.BlockSpec((1, BLK, EMB), lambda i: (1, i, 0)),
                     blk]
    else:
        in_specs += [blk]
    if not last:
        in_specs += [blk]
    in_specs += [blk, blk, wspec, bspec, wspec, bspec]
    out_specs = []
    out_shape = []
    if need_dagg:
        out_specs += [blk]
        out_shape += [jax.ShapeDtypeStruct((N, EMB), jnp.float32)]
    out_specs += [wspec, bspec, wspec, bspec]
    out_shape += [jax.ShapeDtypeStruct((EMB, EMB), jnp.float32),
                  jax.ShapeDtypeStruct((1, EMB), jnp.float32),
                  jax.ShapeDtypeStruct((EMB, EMB), jnp.float32),
                  jax.ShapeDtypeStruct((1, EMB), jnp.float32)]
    return pl.pallas_call(
        body,
        grid=(NB,),
        in_specs=in_specs,
        out_specs=out_specs,
        out_shape=out_shape,
        scratch_shapes=[pltpu.VMEM((EMB, EMB), jnp.float32),
                        pltpu.VMEM((1, EMB), jnp.float32),
                        pltpu.VMEM((EMB, EMB), jnp.float32),
                        pltpu.VMEM((1, EMB), jnp.float32)],
    )


@functools.lru_cache(maxsize=None)
def _make_head(N):
    NB = N // BLK

    def body(h_ref, b_ref, y_ref, wgt_ref, bg_ref,
             loss_ref, ds_ref, wgtn_ref, bgn_ref, sums, cnts):
        i = pl.program_id(0)

        @pl.when(i == 0)
        def _init():
            sums[...] = jnp.zeros((EMB, EMB), jnp.float32)
            cnts[...] = jnp.zeros((EMB, EMB), jnp.float32)

        bids = b_ref[0, 0]
        lane = lax.broadcasted_iota(jnp.int32, (BLK, EMB), 1)
        oh = (lane == bids[:, None]).astype(jnp.float32)
        sums[...] += _dot(oh, h_ref[...], 0, 0)
        cnts[...] += _dot(oh, jnp.ones((BLK, EMB), jnp.float32), 0, 0)

        @pl.when(i == NB - 1)
        def _finish():
            cm = jnp.maximum(cnts[...], 1.0)
            pooled = sums[...] / cm
            wgt = wgt_ref[...]
            pred = jnp.sum(pooled * wgt, axis=1, keepdims=True) + bg_ref[...]
            y = y_ref[...]
            t = (y + 1.0) * 0.5
            valid = (y * y > 1e-5).astype(jnp.float32)
            lm = (jnp.maximum(pred, 0.0) - pred * t
                  + jnp.log1p(jnp.exp(-jnp.abs(pred))))
            vs = jnp.sum(valid)
            loss_ref[...] = jnp.reshape(jnp.sum(lm * valid) / vs, (1, 1))
            dpred = (jax.nn.sigmoid(pred) - t) * valid / vs
            ds_ref[...] = dpred * wgt / cm
            wgtn_ref[...] = wgt - LR * jnp.sum(pooled * dpred, axis=0,
                                               keepdims=True)
            bgn_ref[...] = bg_ref[...] - LR * jnp.sum(dpred)

    one = pl.BlockSpec((1, 1), lambda i: (0, 0))
    emb2 = pl.BlockSpec((EMB, EMB), lambda i: (0, 0))
    return pl.pallas_call(
        body,
        grid=(NB,),
        in_specs=[
            pl.BlockSpec((BLK, EMB), lambda i: (i, 0)),
            pl.BlockSpec((1, 1, BLK), lambda i: (i, 0, 0)),
            pl.BlockSpec((EMB, 1), lambda i: (0, 0)),
            pl.BlockSpec((1, EMB), lambda i: (0, 0)),
            one,
        ],
        out_specs=[one, emb2, pl.BlockSpec((1, EMB), lambda i: (0, 0)), one],
        out_shape=[jax.ShapeDtypeStruct((1, 1), jnp.float32),
                   jax.ShapeDtypeStruct((EMB, EMB), jnp.float32),
                   jax.ShapeDtypeStruct((1, EMB), jnp.float32),
                   jax.ShapeDtypeStruct((1, 1), jnp.float32)],
        scratch_shapes=[pltpu.VMEM((EMB, EMB), jnp.float32),
                        pltpu.VMEM((EMB, EMB), jnp.float32)],
    )


@functools.lru_cache(maxsize=None)
def _make_expand(N):
    NB = N // BLK

    def body(ds_ref, b_ref, dh_ref):
        bids = b_ref[0, 0]
        lane = lax.broadcasted_iota(jnp.int32, (BLK, EMB), 1)
        oh = (lane == bids[:, None]).astype(jnp.float32)
        dh_ref[...] = _dot(oh, ds_ref[...], 1, 0)

    return pl.pallas_call(
        body,
        grid=(NB,),
        in_specs=[
            pl.BlockSpec((EMB, EMB), lambda i: (0, 0)),
            pl.BlockSpec((1, 1, BLK), lambda i: (i, 0, 0)),
        ],
        out_specs=pl.BlockSpec((BLK, EMB), lambda i: (i, 0)),
        out_shape=jax.ShapeDtypeStruct((N, EMB), jnp.float32),
    )


def _forward(x, src, dst, batch3, fw, save):
    W1s, b1s, W2s, b2s = fw[0], fw[1], fw[2], fw[3]
    N = x.shape[0]
    h = x
    aggs, rs, hs = [], [], [h]
    for l in range(NLAYER):
        P = _sc_spmm(h, src, dst)
        agg, r, hn = _make_fwd(N, l == NLAYER - 1)(
            P, P, h, W1s[l], b1s[l], W2s[l], b2s[l])
        if save:
            aggs.append(agg)
            rs.append(r)
            hs.append(hn)
        h = hn
    return h, aggs, rs, hs


def kernel(x_spt, edge_index_spt, batch_spt, y_spt,
           x_qry, edge_index_qry, batch_qry, y_qry,
           W1, b1, W2, b2, Wg, bg):
    N = x_spt.shape[0]
    NB = N // BLK
    src_s = edge_index_spt[0].astype(jnp.int32)
    dst_s = edge_index_spt[1].astype(jnp.int32)
    src_q = edge_index_qry[0].astype(jnp.int32)
    dst_q = edge_index_qry[1].astype(jnp.int32)
    batch_s3 = batch_spt.astype(jnp.int32).reshape(NB, 1, BLK)
    batch_q3 = batch_qry.astype(jnp.int32).reshape(NB, 1, BLK)
    y_s = jnp.pad(y_spt, (0, EMB - y_spt.shape[0])).reshape(EMB, 1)
    y_q = jnp.pad(y_qry, (0, EMB - y_qry.shape[0])).reshape(EMB, 1)

    fW1 = [W1[l] for l in range(NLAYER)]
    fb1 = [b1[l].reshape(1, EMB) for l in range(NLAYER)]
    fW2 = [W2[l] for l in range(NLAYER)]
    fb2 = [b2[l].reshape(1, EMB) for l in range(NLAYER)]
    fwgT = Wg.reshape(1, EMB)
    fbg = bg.reshape(1, 1)

    for _ in range(NSTEP):
        h, aggs, rs, hs = _forward(x_spt, src_s, dst_s, batch_s3,
                                   (fW1, fb1, fW2, fb2), save=True)
        _, d_sums, fwgT_new, fbg_new = _make_head(N)(h, batch_s3, y_s, fwgT, fbg)
        dh = _make_expand(N)(d_sums, batch_s3)
        nW1 = [None] * NLAYER
        nb1 = [None] * NLAYER
        nW2 = [None] * NLAYER
        nb2 = [None] * NLAYER
        dprev = None
        Q = None
        for l in range(NLAYER - 1, -1, -1):
            last = l == NLAYER - 1
            need_dagg = l > 0
            bwd = _make_bwd(N, last, not last, need_dagg)
            args = []
            if last:
                args += [dh]
            else:
                args += [Q, Q, dprev, hs[l + 1]]
            args += [rs[l], aggs[l], fW1[l], fb1[l], fW2[l], fb2[l]]
            outs = bwd(*args)
            if need_dagg:
                dagg = outs[0]
                outs = outs[1:]
                Q = _sc_spmm(dagg, dst_s, src_s)
                dprev = dagg
            nW1[l], nb1[l], nW2[l], nb2[l] = outs
        fW1, fb1, fW2, fb2 = nW1, nb1, nW2, nb2
        fwgT, fbg = fwgT_new, fbg_new

    h, _, _, _ = _forward(x_qry, src_q, dst_q, batch_q3,
                          (fW1, fb1, fW2, fb2), save=False)
    loss, _, _, _ = _make_head(N)(h, batch_q3, y_q, fwgT, fbg)
    return loss[0, 0]
